# baseline (speedup 1.0000x reference)
import functools

import jax
import jax.numpy as jnp
from jax import lax
from jax.experimental import pallas as pl
from jax.experimental.pallas import tpu as pltpu
from jax.experimental.pallas import tpu_sc as plsc

DIM = 4096
BATCH = 4096
L = 16

NC = 2
NS = 16
NW = NC * NS
ROWS_PER_W = BATCH // NW
RB = 8
NB = ROWS_PER_W // RB
NP = NB // 2
W1 = 1664
W2 = DIM - W1

_mesh = plsc.VectorSubcoreMesh(core_axis_name="c", subcore_axis_name="s")


@functools.partial(
    pl.kernel,
    out_type=jax.ShapeDtypeStruct((BATCH, DIM), jnp.float32),
    mesh=_mesh,
    scratch_types=[
        pltpu.VMEM((DIM,), jnp.int32),
        pltpu.VMEM((RB, DIM), jnp.float32),
        pltpu.VMEM((RB, DIM), jnp.float32),
        pltpu.VMEM((RB, W1), jnp.float32),
        pltpu.VMEM((RB, W2), jnp.float32),
        pltpu.VMEM_SHARED((NS, 2, RB, W1), jnp.float32),
        pltpu.SemaphoreType.DMA,
        pltpu.SemaphoreType.DMA,
        pltpu.SemaphoreType.DMA,
        pltpu.SemaphoreType.DMA,
        pltpu.SemaphoreType.DMA,
        pltpu.SemaphoreType.DMA,
        pltpu.SemaphoreType.DMA,
    ],
    compiler_params=pltpu.CompilerParams(
        use_tc_tiling_on_sc=True, needs_layout_passes=False
    ),
)
def _permute(x_hbm, perm_hbm, out_hbm, perm_v, in0, in1, outa, outb, sp,
             si0, si1, sx0, sx1, sh0, sh1, sob):
    wid = lax.axis_index("s") * NC + lax.axis_index("c")
    sid = lax.axis_index("s")
    row0 = wid * ROWS_PER_W

    ins = (in0, in1)
    sin = (si0, si1)
    sxb = (sx0, sx1)
    shb = (sh0, sh1)

    def start_in(b, slot):
        pltpu.make_async_copy(
            x_hbm.at[pl.ds(row0 + b * RB, RB)], ins[slot], sin[slot]).start()

    def wait_in(slot):
        pltpu.make_async_copy(
            x_hbm.at[pl.ds(0, RB)], ins[slot], sin[slot]).wait()

    def start_xbar(p):
        pltpu.make_async_copy(outa, sp.at[sid, p], sxb[p]).start()

    def wait_xbar(p):
        pltpu.make_async_copy(outa, sp.at[sid, p], sxb[p]).wait()

    def start_hbm(b, p):
        pltpu.make_async_copy(
            sp.at[sid, p],
            out_hbm.at[pl.ds(row0 + b * RB, RB), pl.ds(0, W1)],
            shb[p]).start()

    def wait_hbm(p):
        pltpu.make_async_copy(
            sp.at[sid, p],
            out_hbm.at[pl.ds(0, RB), pl.ds(0, W1)],
            shb[p]).wait()

    def start_outb(b):
        pltpu.make_async_copy(
            outb,
            out_hbm.at[pl.ds(row0 + b * RB, RB), pl.ds(W1, W2)],
            sob).start()

    def wait_outb():
        pltpu.make_async_copy(
            outb,
            out_hbm.at[pl.ds(0, RB), pl.ds(W1, W2)],
            sob).wait()

    def gather_piece(src, dst, lo, width):
        @plsc.parallel_loop(lo, lo + width, step=L, unroll=2)
        def _jloop(j):
            pv = perm_v[pl.ds(j, L)]
            for r in range(RB):
                rsel = jnp.full((L,), r, jnp.int32)
                dst[r, pl.ds(j - lo, L)] = plsc.load_gather(src, [rsel, pv])

    start_in(0, 0)
    start_in(1, 1)
    pltpu.sync_copy(perm_hbm, perm_v)

    def do_block(b, slot, have_prev, have_prev2):
        wait_in(slot)
        if have_prev:
            wait_xbar(1 - slot)
            start_hbm(b - 1, 1 - slot)
        gather_piece(ins[slot], outa, 0, W1)
        if have_prev2:
            wait_hbm(slot)
        start_xbar(slot)
        if have_prev:
            wait_outb()
        gather_piece(ins[slot], outb, W1, W2)
        start_outb(b)

    def pair_body(k, carry):
        b0 = 2 * k

        @pl.when(k == 0)
        def _():
            do_block(b0, 0, have_prev=False, have_prev2=False)

        @pl.when(k > 0)
        def _():
            do_block(b0, 0, have_prev=True, have_prev2=True)

        @pl.when(k < NP - 1)
        def _():
            start_in(b0 + 2, 0)

        @pl.when(k == 0)
        def _():
            do_block(b0 + 1, 1, have_prev=True, have_prev2=False)

        @pl.when(k > 0)
        def _():
            do_block(b0 + 1, 1, have_prev=True, have_prev2=True)

        @pl.when(k < NP - 1)
        def _():
            start_in(b0 + 3, 1)
        return carry

    lax.fori_loop(0, NP, pair_body, 0)
    wait_xbar(1)
    start_hbm(NB - 1, 1)
    wait_hbm(0)
    wait_hbm(1)
    wait_outb()


def kernel(x, perm):
    return _permute(x, perm)


# speedup vs baseline: 1.0270x; 1.0270x over previous
import functools

import jax
import jax.numpy as jnp
from jax import lax
from jax.experimental import pallas as pl
from jax.experimental.pallas import tpu as pltpu
from jax.experimental.pallas import tpu_sc as plsc

DIM = 4096
BATCH = 4096
L = 16

NC = 2
NS = 16
NW = NC * NS
ROWS_PER_W = BATCH // NW
RB = 8
NB = ROWS_PER_W // RB
NP = NB // 2
HD = DIM // 2

_mesh = plsc.VectorSubcoreMesh(core_axis_name="c", subcore_axis_name="s")


@functools.partial(
    pl.kernel,
    out_type=jax.ShapeDtypeStruct((BATCH, DIM), jnp.float32),
    mesh=_mesh,
    scratch_types=[
        pltpu.VMEM((DIM,), jnp.int32),
        pltpu.VMEM((RB, DIM), jnp.float32),
        pltpu.VMEM((RB, DIM), jnp.float32),
        pltpu.VMEM((RB, HD), jnp.float32),
        pltpu.VMEM((RB, HD), jnp.float32),
        pltpu.SemaphoreType.DMA,
        pltpu.SemaphoreType.DMA,
        pltpu.SemaphoreType.DMA,
        pltpu.SemaphoreType.DMA,
    ],
    compiler_params=pltpu.CompilerParams(
        use_tc_tiling_on_sc=True, needs_layout_passes=False,
        skip_device_barrier=True,
    ),
)
def _permute(x_hbm, perm_hbm, out_hbm, perm_v, in0, in1, outa, outb,
             si0, si1, soa, sob):
    wid = lax.axis_index("s") * NC + lax.axis_index("c")
    row0 = wid * ROWS_PER_W

    ins = (in0, in1)
    outs = (outa, outb)
    sin = (si0, si1)
    sout = (soa, sob)

    def start_in(b, slot):
        pltpu.make_async_copy(
            x_hbm.at[pl.ds(row0 + b * RB, RB)], ins[slot], sin[slot]).start()

    def wait_in(slot):
        pltpu.make_async_copy(
            x_hbm.at[pl.ds(0, RB)], ins[slot], sin[slot]).wait()

    def start_out(b, h):
        pltpu.make_async_copy(
            outs[h],
            out_hbm.at[pl.ds(row0 + b * RB, RB), pl.ds(h * HD, HD)],
            sout[h]).start()

    def wait_out(h):
        pltpu.make_async_copy(
            outs[h],
            out_hbm.at[pl.ds(0, RB), pl.ds(h * HD, HD)],
            sout[h]).wait()

    def gather_half(src, h):
        @plsc.parallel_loop(h * HD, (h + 1) * HD, step=L, unroll=1)
        def _jloop(j):
            pv = perm_v[pl.ds(j, L)]
            for r in range(RB):
                rsel = jnp.full((L,), r, jnp.int32)
                outs[h][r, pl.ds(j - h * HD, L)] = plsc.load_gather(
                    src, [rsel, pv])

    start_in(0, 0)
    start_in(1, 1)
    pltpu.sync_copy(perm_hbm, perm_v)

    def pair_body(k, carry):
        b0 = 2 * k
        wait_in(0)
        for h in range(2):
            @pl.when(k > 0)
            def _():
                wait_out(h)
            gather_half(ins[0], h)
            start_out(b0, h)

        @pl.when(k < NP - 1)
        def _():
            start_in(b0 + 2, 0)

        wait_in(1)
        for h in range(2):
            wait_out(h)
            gather_half(ins[1], h)
            start_out(b0 + 1, h)

        @pl.when(k < NP - 1)
        def _():
            start_in(b0 + 3, 1)
        return carry

    lax.fori_loop(0, NP, pair_body, 0)
    wait_out(0)
    wait_out(1)


def kernel(x, perm):
    return _permute(x, perm)
